# baseline (speedup 1.0000x reference)
import dataclasses
import functools

import jax
import jax.numpy as jnp
from jax.experimental import pallas as pl
from jax.experimental.pallas import tpu as pltpu
from jax.experimental.pallas import tpu_sc as plsc

QK_DIM = 64
TOPK = 16
SCALE = QK_DIM ** (-0.5)
SEQ = 1024
ROW_BLOCK = 256

NEG_INF = float("-inf")

TC_BATCHES = 12
SC_SLICES = 4


def _fused_kernel(q_ref, k_ref, w_ref, i_ref):
    q = q_ref[0]
    k = k_ref[0]
    logits = jax.lax.dot_general(
        q * SCALE,
        k,
        (((1,), (1,)), ((), ())),
        preferred_element_type=jnp.float32,
    )

    half = SEQ // 2
    a = logits[:, :half]
    b = logits[:, half:]
    colh = jax.lax.broadcasted_iota(jnp.int32, (ROW_BLOCK, half), 1).astype(
        jnp.float32
    )
    o = b > a
    z = jnp.where(o, b, a)
    w = jnp.where(o, a, b)
    RAD = 2048.0
    kp = jnp.where(o, (colh + half) + colh / RAD, colh + (colh + half) / RAD)

    for t in range(TOPK):
        m = jnp.max(z, axis=1, keepdims=True)
        cand = jnp.where(z >= m, kp, RAD)
        pidx = jnp.min(cand, axis=1, keepdims=True)
        idx = jnp.floor(pidx)
        w_ref[0, :, t : t + 1] = m
        i_ref[0, :, t : t + 1] = idx.astype(jnp.int32)
        sel = cand == pidx
        swapped = (pidx - idx) * RAD + idx / RAD
        z = jnp.where(sel, w, z)
        kp = jnp.where(sel, swapped, kp)
        w = jnp.where(sel, NEG_INF, w)

    v = w_ref[0]
    e = jnp.exp(v - v[:, :1])
    w_ref[0] = e / jnp.sum(e, axis=1, keepdims=True)


def _fused(q, k):
    n = q.shape[0]
    return pl.pallas_call(
        _fused_kernel,
        grid=(n, SEQ // ROW_BLOCK),
        in_specs=[
            pl.BlockSpec((1, ROW_BLOCK, QK_DIM), lambda b, r: (b, r, 0)),
            pl.BlockSpec((1, SEQ, QK_DIM), lambda b, r: (b, 0, 0)),
        ],
        out_specs=[
            pl.BlockSpec((1, ROW_BLOCK, TOPK), lambda b, r: (b, r, 0)),
            pl.BlockSpec((1, ROW_BLOCK, TOPK), lambda b, r: (b, r, 0)),
        ],
        out_shape=[
            jax.ShapeDtypeStruct((n, SEQ, TOPK), jnp.float32),
            jax.ShapeDtypeStruct((n, SEQ, TOPK), jnp.int32),
        ],
        compiler_params=pltpu.CompilerParams(
            dimension_semantics=("arbitrary", "arbitrary"),
        ),
    )(q, k)


def _logits_kernel(q_ref, k_ref, o_ref):
    o_ref[0] = jax.lax.dot_general(
        q_ref[0] * SCALE,
        k_ref[0],
        (((1,), (1,)), ((), ())),
        preferred_element_type=jnp.float32,
    )


def _logits(q, k):
    n = q.shape[0]
    return pl.pallas_call(
        _logits_kernel,
        grid=(n,),
        in_specs=[
            pl.BlockSpec((1, SEQ, QK_DIM), lambda b: (b, 0, 0)),
            pl.BlockSpec((1, SEQ, QK_DIM), lambda b: (b, 0, 0)),
        ],
        out_specs=pl.BlockSpec((1, SEQ, SEQ), lambda b: (b, 0, 0)),
        out_shape=jax.ShapeDtypeStruct((n, SEQ, SEQ), jnp.float32),
    )(q, k)


def _sc_topk(x):
    rows = x.shape[0]
    mesh = plsc.VectorSubcoreMesh(core_axis_name="c", subcore_axis_name="s")

    cp = pltpu.CompilerParams()
    if "needs_layout_passes" in pltpu.CompilerParams.__dataclass_fields__:
        cp = dataclasses.replace(cp, needs_layout_passes=False)

    @pl.kernel(
        out_type=[
            jax.ShapeDtypeStruct((rows, TOPK), jnp.float32),
            jax.ShapeDtypeStruct((rows, TOPK), jnp.int32),
        ],
        mesh=mesh,
        compiler_params=cp,
    )
    def sck(x_hbm, w_hbm, i_hbm):
        def body(x_vmem, w_vmem, i_vmem):
            xr = x_vmem.at[0]
            cur_v, cur_i = plsc.sort_key_val(
                xr[pl.ds(0, TOPK)],
                jax.lax.iota(jnp.int32, TOPK),
                descending=True,
            )
            for ch in range(1, SEQ // TOPK):
                v = xr[pl.ds(ch * TOPK, TOPK)]
                ci = jax.lax.iota(jnp.int32, TOPK) + ch * TOPK
                sv, si = plsc.sort_key_val(v, ci)
                mv = jnp.maximum(cur_v, sv)
                mi = jnp.where(cur_v >= sv, cur_i, si)
                cur_v, cur_i = plsc.sort_key_val(mv, mi, descending=True)
            m = jnp.max(cur_v)
            e = jnp.exp(cur_v - m)
            w_vmem[0, :] = e / jnp.sum(e)
            i_vmem[0, :] = cur_i

        pltpu.emit_pipeline(
            body,
            grid=(rows,),
            in_specs=[pl.BlockSpec((1, SEQ), lambda r: (r, 0))],
            out_specs=[
                pl.BlockSpec((1, TOPK), lambda r: (r, 0)),
                pl.BlockSpec((1, TOPK), lambda r: (r, 0)),
            ],
            core_axis_name=("c", "s"),
            dimension_semantics=(pltpu.PARALLEL,),
        )(x_hbm, w_hbm, i_hbm)

    return sck(x)


@jax.jit
def kernel(query, key):
    n, s, c = query.shape
    n_sc = n - TC_BATCHES
    step = n_sc // SC_SLICES
    ws, ixs = [], []
    for p in range(SC_SLICES):
        qp = query[p * step : (p + 1) * step]
        kp = key[p * step : (p + 1) * step]
        logits = _logits(qp, kp)
        w, ix = _sc_topk(logits.reshape(step * s, s))
        ws.append(w.reshape(step, s, TOPK))
        ixs.append(ix.reshape(step, s, TOPK))
    wf, ixf = _fused(query[n_sc:], key[n_sc:])
    ws.append(wf)
    ixs.append(ixf)
    return jnp.concatenate(ws, axis=0), jnp.concatenate(ixs, axis=0)
